# baseline (speedup 1.0000x reference)
import jax
import jax.numpy as jnp
from jax import lax
from jax.experimental import pallas as pl
from jax.experimental.pallas import tpu as pltpu
import jax.experimental.pallas.tpu_sc as plsc

N = 10000
D = 256
E = 160000
NPAD = 10240
BLK = 320
NT = 16
NSC = 2
NW = NSC * NT
CAP = 7936
CAP2 = 8064
PADROW = 10200
NV = 8

ESEC, EROW, EK = 50, 25, 128
G = 48
UNR = 4

_mesh = plsc.VectorSubcoreMesh(core_axis_name="c", subcore_axis_name="s")
_sc_params = pltpu.CompilerParams(needs_layout_passes=False)


def _scan_body(src_ref, dst_ref, degh_out, sl_out, dl_out, cnt_out,
               s0, s1, d0, d1, slist, dlist, fold, cbuf, semA, semB,
               *hists):
    c = lax.axis_index("c")
    s = lax.axis_index("s")
    w = c * NT + s
    rowbase = w * BLK
    iota = lax.iota(jnp.int32, 16)

    def zbody(r, carry):
        hists[0][pl.ds(r * 16, 16)] = jnp.zeros((16,), jnp.float32)
        return carry

    lax.fori_loop(0, BLK, zbody, 0)

    def fbody(i, carry):
        slist[pl.ds(i * 16, 16)] = jnp.full((16,), PADROW, jnp.int32)
        dlist[pl.ds(i * 16, 16)] = jnp.zeros((16,), jnp.int32)
        return carry

    lax.fori_loop(0, CAP2 // 16, fbody, 0)

    def scan(sbuf, dbuf, wp):
        def sbody(j, wp2):
            for v in range(NV):
                d = dbuf[j, pl.ds(v * 16, 16)]
                sv = sbuf[j, pl.ds(v * 16, 16)]
                m = (d >= rowbase) & (d < rowbase + BLK)
                dl = jnp.where(m, d - rowbase, 0)
                mi = m.astype(jnp.int32)
                pos = wp2 + plsc.cumsum(mi) - mi
                pos = jnp.minimum(pos, CAP - 1)
                plsc.store_scatter(slist, [pos], sv, mask=m)
                plsc.store_scatter(dlist, [pos], dl, mask=m)
                wp2 = wp2 + plsc.all_reduce_population_count(m)
            return wp2
        return lax.fori_loop(0, EROW, sbody, wp)

    pltpu.async_copy(src_ref.at[0], s0, semA)
    pltpu.async_copy(dst_ref.at[0], d0, semA)

    def pairbody(i, wp):
        pltpu.async_copy(src_ref.at[2 * i + 1], s1, semB)
        pltpu.async_copy(dst_ref.at[2 * i + 1], d1, semB)
        pltpu.make_async_copy(src_ref.at[0], s0, semA).wait()
        pltpu.make_async_copy(dst_ref.at[0], d0, semA).wait()
        wp = scan(s0, d0, wp)

        @pl.when(2 * i + 2 < ESEC)
        def _():
            pltpu.async_copy(src_ref.at[2 * i + 2], s0, semA)
            pltpu.async_copy(dst_ref.at[2 * i + 2], d0, semA)

        pltpu.make_async_copy(src_ref.at[0], s1, semB).wait()
        pltpu.make_async_copy(dst_ref.at[0], d1, semB).wait()
        return scan(s1, d1, wp)

    wp = lax.fori_loop(0, ESEC // 2, pairbody,
                       jnp.zeros((16,), jnp.int32))
    count = jnp.max(wp)

    def hbody(g, carry):
        dl = dlist[pl.ds(g * 16, 16)]
        m = (g * 16 + iota) < count
        hpos = dl * 16 + iota
        old = plsc.load_gather(hists[0], [hpos])
        plsc.store_scatter(hists[0], [hpos],
                           old + jnp.where(m, 1.0, 0.0))
        return carry

    lax.fori_loop(0, (count + 15) // 16, hbody, 0)

    def foldbody(g, carry):
        fold[pl.ds(g * 16, 16)] = hists[0][pl.ds(g * 16, 16)]
        return carry

    lax.fori_loop(0, BLK, foldbody, 0)
    cbuf[pl.ds(0, 16)] = jnp.minimum(wp, CAP)
    pltpu.sync_copy(fold, degh_out.at[pl.ds(rowbase * 16, BLK * 16)])
    pltpu.sync_copy(slist, sl_out.at[w])
    pltpu.sync_copy(dlist, dl_out.at[w])
    pltpu.sync_copy(cbuf, cnt_out.at[w])


_scan_call = pl.kernel(
    _scan_body,
    out_type=[
        jax.ShapeDtypeStruct((NPAD * 16,), jnp.float32),
        jax.ShapeDtypeStruct((NW, CAP2), jnp.int32),
        jax.ShapeDtypeStruct((NW, CAP2), jnp.int32),
        jax.ShapeDtypeStruct((NW, 16), jnp.int32),
    ],
    mesh=_mesh,
    compiler_params=_sc_params,
    scratch_types=[
        pltpu.VMEM((EROW, EK), jnp.int32),
        pltpu.VMEM((EROW, EK), jnp.int32),
        pltpu.VMEM((EROW, EK), jnp.int32),
        pltpu.VMEM((EROW, EK), jnp.int32),
        pltpu.VMEM((CAP2,), jnp.int32),
        pltpu.VMEM((CAP2,), jnp.int32),
        pltpu.VMEM((BLK * 16,), jnp.float32),
        pltpu.VMEM((16,), jnp.int32),
        pltpu.SemaphoreType.DMA,
        pltpu.SemaphoreType.DMA,
    ] + [pltpu.VMEM((BLK * 16,), jnp.float32)] * 1,
)


def _k2_body(x_ref, dh_ref, g_ref, bt_ref, w_ref, o_ref):
    xb = x_ref[...]
    mu = jnp.mean(xb, axis=1, keepdims=True)
    xc = xb - mu
    var = jnp.mean(xc * xc, axis=1, keepdims=True)
    h = xc * lax.rsqrt(var + 1e-5) * g_ref[...] + bt_ref[...]
    h = jnp.where(h >= 0, h, 0.01 * h)
    hw = jnp.dot(h, w_ref[...], preferred_element_type=jnp.float32)
    deg = jnp.sum(dh_ref[...], axis=1, keepdims=True) + 1.0
    o_ref[...] = hw * lax.rsqrt(deg)


_BN = 256


def _k2_call(xpad, degh, g2, bt2, W):
    grid = (NPAD // _BN,)
    return pl.pallas_call(
        _k2_body,
        grid=grid,
        in_specs=[
            pl.BlockSpec((_BN, D), lambda i: (i, 0)),
            pl.BlockSpec((_BN, 16), lambda i: (i, 0)),
            pl.BlockSpec((1, D), lambda i: (0, 0)),
            pl.BlockSpec((1, D), lambda i: (0, 0)),
            pl.BlockSpec((D, D), lambda i: (0, 0)),
        ],
        out_specs=pl.BlockSpec((_BN, D), lambda i: (i, 0)),
        out_shape=jax.ShapeDtypeStruct((NPAD, D), jnp.float32),
    )(xpad, degh, g2, bt2, W)


def _sc_body(hw_ref, sl_ref, dl_ref, cnt_ref, out_ref,
             slist, dlist, cbuf, rows0, rows1, sem0, sem1, *accs):
    c = lax.axis_index("c")
    s = lax.axis_index("s")
    w = c * NT + s
    rowbase = w * BLK
    iota = lax.iota(jnp.int32, 16)

    pltpu.sync_copy(sl_ref.at[w], slist)
    pltpu.sync_copy(dl_ref.at[w], dlist)
    pltpu.sync_copy(cnt_ref.at[w], cbuf)

    def zbody(r, carry):
        for v in range(D // 16):
            accs[v][pl.ds(r * 16, 16)] = jnp.zeros((16,), jnp.float32)
        return carry

    lax.fori_loop(0, BLK, zbody, 0)

    count = jnp.max(cbuf[pl.ds(0, 16)])
    npair = (count + (2 * G - 1)) // (2 * G)

    def accum(ch, rbuf):
        def abody(q, carry):
            for u in range(UNR):
                r = q * UNR + u
                dlx = plsc.load_gather(
                    dlist, [jnp.full((16,), ch * G + r, jnp.int32)])
                ds = jnp.max(dlx) * 16
                for v in range(D // 16):
                    vals = rbuf[r, pl.ds(v * 16, 16)]
                    accs[v][pl.ds(ds, 16)] = accs[v][pl.ds(ds, 16)] + vals
            return carry
        lax.fori_loop(0, G // UNR, abody, 0)

    def start(ch, rbuf, sem):
        pltpu.async_copy(hw_ref.at[slist.at[pl.ds(ch * G, G)]], rbuf, sem)

    start(0, rows0, sem0)

    def gpair(i, carry):
        start(2 * i + 1, rows1, sem1)
        pltpu.make_async_copy(hw_ref.at[pl.ds(0, G)], rows0, sem0).wait()
        accum(2 * i, rows0)

        @pl.when(2 * i + 2 < 2 * npair)
        def _():
            start(2 * i + 2, rows0, sem0)

        pltpu.make_async_copy(hw_ref.at[pl.ds(0, G)], rows1, sem1).wait()
        accum(2 * i + 1, rows1)
        return carry

    lax.fori_loop(0, npair, gpair, 0)

    for k in range(BLK // 40):
        def dbody(rr, carry):
            r = k * 40 + rr
            for v in range(D // 16):
                rows0[rr, pl.ds(v * 16, 16)] = accs[v][pl.ds(r * 16, 16)]
            return carry

        lax.fori_loop(0, 40, dbody, 0)
        pltpu.sync_copy(rows0.at[pl.ds(0, 40)],
                        out_ref.at[pl.ds(rowbase + k * 40, 40)])


_sc_call = pl.kernel(
    _sc_body,
    out_type=jax.ShapeDtypeStruct((NPAD, D), jnp.float32),
    mesh=_mesh,
    compiler_params=_sc_params,
    scratch_types=[
        pltpu.VMEM((CAP2,), jnp.int32),
        pltpu.VMEM((CAP2,), jnp.int32),
        pltpu.VMEM((16,), jnp.int32),
        pltpu.VMEM((G, D), jnp.float32),
        pltpu.VMEM((G, D), jnp.float32),
        pltpu.SemaphoreType.DMA,
        pltpu.SemaphoreType.DMA,
    ] + [pltpu.VMEM((BLK * 16,), jnp.float32)] * (D // 16),
)


def _k4_body(x_ref, a_ref, hw_ref, dh_ref, b_ref, o_ref):
    deg = jnp.sum(dh_ref[...], axis=1, keepdims=True) + 1.0
    dinv = lax.rsqrt(deg)
    o_ref[...] = (x_ref[...] + dinv * (a_ref[...] + hw_ref[...])
                  + b_ref[...])


def _k4_call(xpad, agg, hw, degh, b2):
    grid = (NPAD // _BN,)
    return pl.pallas_call(
        _k4_body,
        grid=grid,
        in_specs=[
            pl.BlockSpec((_BN, D), lambda i: (i, 0)),
            pl.BlockSpec((_BN, D), lambda i: (i, 0)),
            pl.BlockSpec((_BN, D), lambda i: (i, 0)),
            pl.BlockSpec((_BN, 16), lambda i: (i, 0)),
            pl.BlockSpec((1, D), lambda i: (0, 0)),
        ],
        out_specs=pl.BlockSpec((_BN, D), lambda i: (i, 0)),
        out_shape=jax.ShapeDtypeStruct((NPAD, D), jnp.float32),
    )(xpad, agg, hw, degh, b2)


@jax.jit
def kernel(x, edge_index, W, b, gamma, beta):
    src = edge_index[0]
    dst = edge_index[1]
    xpad = jnp.pad(x, ((0, NPAD - N), (0, 0)))
    src4 = src.reshape(ESEC, EROW, EK)
    dst4 = dst.reshape(ESEC, EROW, EK)

    degh, sl, dl, cnt = _scan_call(src4, dst4)
    degh = degh.reshape(NPAD, 16)
    hw = _k2_call(xpad, degh, gamma[None], beta[None], W)
    agg = _sc_call(hw, sl, dl, cnt)
    out = _k4_call(xpad, agg, hw, degh, b[None])
    return out[:N]


# speedup vs baseline: 1.0142x; 1.0142x over previous
import jax
import jax.numpy as jnp
from jax import lax
from jax.experimental import pallas as pl
from jax.experimental.pallas import tpu as pltpu
import jax.experimental.pallas.tpu_sc as plsc

N = 10000
D = 256
E = 160000
NPAD = 10240
BLK = 320
NT = 16
NSC = 2
NW = NSC * NT
CAP = 7936
CAP2 = 8064
PADROW = 10200
NV = 8

ESEC, EROW, EK = 50, 25, 128
G = 48
UNR = 4

_mesh = plsc.VectorSubcoreMesh(core_axis_name="c", subcore_axis_name="s")
_sc_params = pltpu.CompilerParams(needs_layout_passes=False)


def _scan_body(src_ref, dst_ref, degh_out, sl_out, dl_out, cnt_out,
               s0, s1, d0, d1, slist, dlist, fold, cbuf, semA, semB,
               *hists):
    c = lax.axis_index("c")
    s = lax.axis_index("s")
    w = c * NT + s
    rowbase = w * BLK
    iota = lax.iota(jnp.int32, 16)

    def zbody(r, carry):
        hists[0][pl.ds(r * 16, 16)] = jnp.zeros((16,), jnp.float32)
        return carry

    lax.fori_loop(0, BLK, zbody, 0)

    def fbody(i, carry):
        slist[pl.ds(i * 16, 16)] = jnp.full((16,), PADROW, jnp.int32)
        dlist[pl.ds(i * 16, 16)] = jnp.zeros((16,), jnp.int32)
        return carry

    lax.fori_loop(0, CAP2 // 16, fbody, 0)

    def scan(sbuf, dbuf, wp):
        def sbody(j, wp2):
            for v in range(NV):
                d = dbuf[j, pl.ds(v * 16, 16)]
                sv = sbuf[j, pl.ds(v * 16, 16)]
                m = (d >= rowbase) & (d < rowbase + BLK)
                dl = jnp.where(m, d - rowbase, 0)
                mi = m.astype(jnp.int32)
                pos = wp2 + plsc.cumsum(mi) - mi
                pos = jnp.minimum(pos, CAP - 1)
                plsc.store_scatter(slist, [pos], sv, mask=m)
                plsc.store_scatter(dlist, [pos], dl, mask=m)
                wp2 = wp2 + plsc.all_reduce_population_count(m)
            return wp2
        return lax.fori_loop(0, EROW, sbody, wp)

    pltpu.async_copy(src_ref.at[0], s0, semA)
    pltpu.async_copy(dst_ref.at[0], d0, semA)

    def pairbody(i, wp):
        pltpu.async_copy(src_ref.at[2 * i + 1], s1, semB)
        pltpu.async_copy(dst_ref.at[2 * i + 1], d1, semB)
        pltpu.make_async_copy(src_ref.at[0], s0, semA).wait()
        pltpu.make_async_copy(dst_ref.at[0], d0, semA).wait()
        wp = scan(s0, d0, wp)

        @pl.when(2 * i + 2 < ESEC)
        def _():
            pltpu.async_copy(src_ref.at[2 * i + 2], s0, semA)
            pltpu.async_copy(dst_ref.at[2 * i + 2], d0, semA)

        pltpu.make_async_copy(src_ref.at[0], s1, semB).wait()
        pltpu.make_async_copy(dst_ref.at[0], d1, semB).wait()
        return scan(s1, d1, wp)

    wp = lax.fori_loop(0, ESEC // 2, pairbody,
                       jnp.zeros((16,), jnp.int32))
    count = jnp.max(wp)

    def hbody(g, carry):
        dl = dlist[pl.ds(g * 16, 16)]
        m = (g * 16 + iota) < count
        hpos = dl * 16 + iota
        old = plsc.load_gather(hists[0], [hpos])
        plsc.store_scatter(hists[0], [hpos],
                           old + jnp.where(m, 1.0, 0.0))
        return carry

    lax.fori_loop(0, (count + 15) // 16, hbody, 0)

    def foldbody(g, carry):
        fold[pl.ds(g * 16, 16)] = hists[0][pl.ds(g * 16, 16)]
        return carry

    lax.fori_loop(0, BLK, foldbody, 0)
    cbuf[pl.ds(0, 16)] = jnp.minimum(wp, CAP)
    pltpu.sync_copy(fold, degh_out.at[pl.ds(rowbase * 16, BLK * 16)])
    pltpu.sync_copy(slist, sl_out.at[w])
    pltpu.sync_copy(dlist, dl_out.at[w])
    pltpu.sync_copy(cbuf, cnt_out.at[w])


_scan_call = pl.kernel(
    _scan_body,
    out_type=[
        jax.ShapeDtypeStruct((NPAD * 16,), jnp.float32),
        jax.ShapeDtypeStruct((NW, CAP2), jnp.int32),
        jax.ShapeDtypeStruct((NW, CAP2), jnp.int32),
        jax.ShapeDtypeStruct((NW, 16), jnp.int32),
    ],
    mesh=_mesh,
    compiler_params=_sc_params,
    scratch_types=[
        pltpu.VMEM((EROW, EK), jnp.int32),
        pltpu.VMEM((EROW, EK), jnp.int32),
        pltpu.VMEM((EROW, EK), jnp.int32),
        pltpu.VMEM((EROW, EK), jnp.int32),
        pltpu.VMEM((CAP2,), jnp.int32),
        pltpu.VMEM((CAP2,), jnp.int32),
        pltpu.VMEM((BLK * 16,), jnp.float32),
        pltpu.VMEM((16,), jnp.int32),
        pltpu.SemaphoreType.DMA,
        pltpu.SemaphoreType.DMA,
    ] + [pltpu.VMEM((BLK * 16,), jnp.float32)] * 1,
)


def _k2_body(x_ref, dh_ref, g_ref, bt_ref, w_ref, o_ref):
    xb = x_ref[...]
    mu = jnp.mean(xb, axis=1, keepdims=True)
    xc = xb - mu
    var = jnp.mean(xc * xc, axis=1, keepdims=True)
    h = xc * lax.rsqrt(var + 1e-5) * g_ref[...] + bt_ref[...]
    h = jnp.where(h >= 0, h, 0.01 * h)
    hw = jnp.dot(h, w_ref[...], preferred_element_type=jnp.float32)
    deg = jnp.sum(dh_ref[...], axis=1, keepdims=True) + 1.0
    o_ref[...] = hw * lax.rsqrt(deg)


_BN = 256


def _k2_call(xpad, degh, g2, bt2, W):
    grid = (NPAD // _BN,)
    return pl.pallas_call(
        _k2_body,
        grid=grid,
        in_specs=[
            pl.BlockSpec((_BN, D), lambda i: (i, 0)),
            pl.BlockSpec((_BN, 16), lambda i: (i, 0)),
            pl.BlockSpec((1, D), lambda i: (0, 0)),
            pl.BlockSpec((1, D), lambda i: (0, 0)),
            pl.BlockSpec((D, D), lambda i: (0, 0)),
        ],
        out_specs=pl.BlockSpec((_BN, D), lambda i: (i, 0)),
        out_shape=jax.ShapeDtypeStruct((NPAD, D), jnp.float32),
    )(xpad, degh, g2, bt2, W)


def _sc_body(hw_ref, sl_ref, dl_ref, cnt_ref, out_ref,
             slist, dlist, cbuf, rows0, rows1, sem0, sem1, *accs):
    c = lax.axis_index("c")
    s = lax.axis_index("s")
    w = c * NT + s
    rowbase = w * BLK
    iota = lax.iota(jnp.int32, 16)

    pltpu.sync_copy(sl_ref.at[w], slist)
    pltpu.sync_copy(dl_ref.at[w], dlist)
    pltpu.sync_copy(cnt_ref.at[w], cbuf)

    def zbody(r, carry):
        for v in range(D // 16):
            accs[v][pl.ds(r * 16, 16)] = jnp.zeros((16,), jnp.float32)
        return carry

    lax.fori_loop(0, BLK, zbody, 0)

    count = jnp.max(cbuf[pl.ds(0, 16)])
    npair = (count + (2 * G - 1)) // (2 * G)

    def accum(ch, rbuf):
        def abody(q, carry):
            for u in range(UNR):
                r = q * UNR + u
                dlx = plsc.load_gather(
                    dlist, [jnp.full((16,), ch * G + r, jnp.int32)])
                pbase = dlx * 16 + iota
                for v in range(D // 16):
                    vals = rbuf[r, pl.ds(v * 16, 16)]
                    old = plsc.load_gather(accs[v], [pbase])
                    plsc.store_scatter(accs[v], [pbase], old + vals)
            return carry
        lax.fori_loop(0, G // UNR, abody, 0)

    def start(ch, rbuf, sem):
        pltpu.async_copy(hw_ref.at[slist.at[pl.ds(ch * G, G)]], rbuf, sem)

    start(0, rows0, sem0)

    def gpair(i, carry):
        start(2 * i + 1, rows1, sem1)
        pltpu.make_async_copy(hw_ref.at[pl.ds(0, G)], rows0, sem0).wait()
        accum(2 * i, rows0)

        @pl.when(2 * i + 2 < 2 * npair)
        def _():
            start(2 * i + 2, rows0, sem0)

        pltpu.make_async_copy(hw_ref.at[pl.ds(0, G)], rows1, sem1).wait()
        accum(2 * i + 1, rows1)
        return carry

    lax.fori_loop(0, npair, gpair, 0)

    for k in range(BLK // 40):
        def dbody(rr, carry):
            r = k * 40 + rr
            for v in range(D // 16):
                rows0[rr, pl.ds(v * 16, 16)] = accs[v][pl.ds(r * 16, 16)]
            return carry

        lax.fori_loop(0, 40, dbody, 0)
        pltpu.sync_copy(rows0.at[pl.ds(0, 40)],
                        out_ref.at[pl.ds(rowbase + k * 40, 40)])


_sc_call = pl.kernel(
    _sc_body,
    out_type=jax.ShapeDtypeStruct((NPAD, D), jnp.float32),
    mesh=_mesh,
    compiler_params=_sc_params,
    scratch_types=[
        pltpu.VMEM((CAP2,), jnp.int32),
        pltpu.VMEM((CAP2,), jnp.int32),
        pltpu.VMEM((16,), jnp.int32),
        pltpu.VMEM((G, D), jnp.float32),
        pltpu.VMEM((G, D), jnp.float32),
        pltpu.SemaphoreType.DMA,
        pltpu.SemaphoreType.DMA,
    ] + [pltpu.VMEM((BLK * 16,), jnp.float32)] * (D // 16),
)


def _k4_body(x_ref, a_ref, hw_ref, dh_ref, b_ref, o_ref):
    deg = jnp.sum(dh_ref[...], axis=1, keepdims=True) + 1.0
    dinv = lax.rsqrt(deg)
    o_ref[...] = (x_ref[...] + dinv * (a_ref[...] + hw_ref[...])
                  + b_ref[...])


def _k4_call(xpad, agg, hw, degh, b2):
    grid = (NPAD // _BN,)
    return pl.pallas_call(
        _k4_body,
        grid=grid,
        in_specs=[
            pl.BlockSpec((_BN, D), lambda i: (i, 0)),
            pl.BlockSpec((_BN, D), lambda i: (i, 0)),
            pl.BlockSpec((_BN, D), lambda i: (i, 0)),
            pl.BlockSpec((_BN, 16), lambda i: (i, 0)),
            pl.BlockSpec((1, D), lambda i: (0, 0)),
        ],
        out_specs=pl.BlockSpec((_BN, D), lambda i: (i, 0)),
        out_shape=jax.ShapeDtypeStruct((NPAD, D), jnp.float32),
    )(xpad, agg, hw, degh, b2)


@jax.jit
def kernel(x, edge_index, W, b, gamma, beta):
    src = edge_index[0]
    dst = edge_index[1]
    xpad = jnp.pad(x, ((0, NPAD - N), (0, 0)))
    src4 = src.reshape(ESEC, EROW, EK)
    dst4 = dst.reshape(ESEC, EROW, EK)

    degh, sl, dl, cnt = _scan_call(src4, dst4)
    degh = degh.reshape(NPAD, 16)
    hw = _k2_call(xpad, degh, gamma[None], beta[None], W)
    agg = _sc_call(hw, sl, dl, cnt)
    out = _k4_call(xpad, agg, hw, degh, b[None])
    return out[:N]


# speedup vs baseline: 1.0395x; 1.0250x over previous
import jax
import jax.numpy as jnp
from jax import lax
from jax.experimental import pallas as pl
from jax.experimental.pallas import tpu as pltpu
import jax.experimental.pallas.tpu_sc as plsc

N = 10000
D = 256
E = 160000
NPAD = 10240
BLK = 320
NT = 16
NSC = 2
NW = NSC * NT
CAP = 7936
CAP2 = 8064
PADROW = 10200
NV = 8

ESEC, EROW, EK = 50, 25, 128
G = 48
UNR = 4

_mesh = plsc.VectorSubcoreMesh(core_axis_name="c", subcore_axis_name="s")
_sc_params = pltpu.CompilerParams(needs_layout_passes=False)


def _scan_body(src_ref, dst_ref, degh_out, sl_out, dl_out, cnt_out,
               s0, s1, d0, d1, slist, dlist, fold, cbuf, semA, semB,
               *hists):
    c = lax.axis_index("c")
    s = lax.axis_index("s")
    w = c * NT + s
    rowbase = w * BLK
    iota = lax.iota(jnp.int32, 16)

    def zbody(r, carry):
        hists[0][pl.ds(r * 16, 16)] = jnp.zeros((16,), jnp.float32)
        return carry

    lax.fori_loop(0, BLK, zbody, 0)

    def fbody(i, carry):
        slist[pl.ds(i * 16, 16)] = jnp.full((16,), PADROW, jnp.int32)
        dlist[pl.ds(i * 16, 16)] = jnp.zeros((16,), jnp.int32)
        return carry

    lax.fori_loop(0, CAP2 // 16, fbody, 0)

    def scan(sbuf, dbuf, wp):
        def sbody(j, wp2):
            for v in range(NV):
                d = dbuf[j, pl.ds(v * 16, 16)]
                sv = sbuf[j, pl.ds(v * 16, 16)]
                m = (d >= rowbase) & (d < rowbase + BLK)
                dl = jnp.where(m, d - rowbase, 0)
                mi = m.astype(jnp.int32)
                pos = wp2 + plsc.cumsum(mi) - mi
                pos = jnp.minimum(pos, CAP - 1)
                plsc.store_scatter(slist, [pos], sv, mask=m)
                plsc.store_scatter(dlist, [pos], dl, mask=m)
                wp2 = wp2 + plsc.all_reduce_population_count(m)
            return wp2
        return plsc.parallel_loop(0, EROW, carry=wp)(sbody)

    pltpu.async_copy(src_ref.at[0], s0, semA)
    pltpu.async_copy(dst_ref.at[0], d0, semA)

    def pairbody(i, wp):
        pltpu.async_copy(src_ref.at[2 * i + 1], s1, semB)
        pltpu.async_copy(dst_ref.at[2 * i + 1], d1, semB)
        pltpu.make_async_copy(src_ref.at[0], s0, semA).wait()
        pltpu.make_async_copy(dst_ref.at[0], d0, semA).wait()
        wp = scan(s0, d0, wp)

        @pl.when(2 * i + 2 < ESEC)
        def _():
            pltpu.async_copy(src_ref.at[2 * i + 2], s0, semA)
            pltpu.async_copy(dst_ref.at[2 * i + 2], d0, semA)

        pltpu.make_async_copy(src_ref.at[0], s1, semB).wait()
        pltpu.make_async_copy(dst_ref.at[0], d1, semB).wait()
        return scan(s1, d1, wp)

    wp = lax.fori_loop(0, ESEC // 2, pairbody,
                       jnp.zeros((16,), jnp.int32))
    count = jnp.max(wp)

    def hbody(g, carry):
        dl = dlist[pl.ds(g * 16, 16)]
        m = (g * 16 + iota) < count
        hpos = dl * 16 + iota
        old = plsc.load_gather(hists[0], [hpos])
        plsc.store_scatter(hists[0], [hpos],
                           old + jnp.where(m, 1.0, 0.0))
        return carry

    lax.fori_loop(0, (count + 15) // 16, hbody, 0)

    def foldbody(g, carry):
        fold[pl.ds(g * 16, 16)] = hists[0][pl.ds(g * 16, 16)]
        return carry

    lax.fori_loop(0, BLK, foldbody, 0)
    cbuf[pl.ds(0, 16)] = jnp.minimum(wp, CAP)
    pltpu.sync_copy(fold, degh_out.at[pl.ds(rowbase * 16, BLK * 16)])
    pltpu.sync_copy(slist, sl_out.at[w])
    pltpu.sync_copy(dlist, dl_out.at[w])
    pltpu.sync_copy(cbuf, cnt_out.at[w])


_scan_call = pl.kernel(
    _scan_body,
    out_type=[
        jax.ShapeDtypeStruct((NPAD * 16,), jnp.float32),
        jax.ShapeDtypeStruct((NW, CAP2), jnp.int32),
        jax.ShapeDtypeStruct((NW, CAP2), jnp.int32),
        jax.ShapeDtypeStruct((NW, 16), jnp.int32),
    ],
    mesh=_mesh,
    compiler_params=_sc_params,
    scratch_types=[
        pltpu.VMEM((EROW, EK), jnp.int32),
        pltpu.VMEM((EROW, EK), jnp.int32),
        pltpu.VMEM((EROW, EK), jnp.int32),
        pltpu.VMEM((EROW, EK), jnp.int32),
        pltpu.VMEM((CAP2,), jnp.int32),
        pltpu.VMEM((CAP2,), jnp.int32),
        pltpu.VMEM((BLK * 16,), jnp.float32),
        pltpu.VMEM((16,), jnp.int32),
        pltpu.SemaphoreType.DMA,
        pltpu.SemaphoreType.DMA,
    ] + [pltpu.VMEM((BLK * 16,), jnp.float32)] * 1,
)


def _k2_body(x_ref, dh_ref, g_ref, bt_ref, w_ref, o_ref):
    xb = x_ref[...]
    mu = jnp.mean(xb, axis=1, keepdims=True)
    xc = xb - mu
    var = jnp.mean(xc * xc, axis=1, keepdims=True)
    h = xc * lax.rsqrt(var + 1e-5) * g_ref[...] + bt_ref[...]
    h = jnp.where(h >= 0, h, 0.01 * h)
    hw = jnp.dot(h, w_ref[...], preferred_element_type=jnp.float32)
    deg = jnp.sum(dh_ref[...], axis=1, keepdims=True) + 1.0
    o_ref[...] = hw * lax.rsqrt(deg)


_BN = 256


def _k2_call(xpad, degh, g2, bt2, W):
    grid = (NPAD // _BN,)
    return pl.pallas_call(
        _k2_body,
        grid=grid,
        in_specs=[
            pl.BlockSpec((_BN, D), lambda i: (i, 0)),
            pl.BlockSpec((_BN, 16), lambda i: (i, 0)),
            pl.BlockSpec((1, D), lambda i: (0, 0)),
            pl.BlockSpec((1, D), lambda i: (0, 0)),
            pl.BlockSpec((D, D), lambda i: (0, 0)),
        ],
        out_specs=pl.BlockSpec((_BN, D), lambda i: (i, 0)),
        out_shape=jax.ShapeDtypeStruct((NPAD, D), jnp.float32),
    )(xpad, degh, g2, bt2, W)


def _sc_body(hw_ref, sl_ref, dl_ref, cnt_ref, out_ref,
             slist, dlist, cbuf, rows0, rows1, sem0, sem1, *accs):
    c = lax.axis_index("c")
    s = lax.axis_index("s")
    w = c * NT + s
    rowbase = w * BLK
    iota = lax.iota(jnp.int32, 16)

    pltpu.sync_copy(sl_ref.at[w], slist)
    pltpu.sync_copy(dl_ref.at[w], dlist)
    pltpu.sync_copy(cnt_ref.at[w], cbuf)

    def zbody(r, carry):
        for v in range(D // 16):
            accs[v][pl.ds(r * 16, 16)] = jnp.zeros((16,), jnp.float32)
        return carry

    lax.fori_loop(0, BLK, zbody, 0)

    count = jnp.max(cbuf[pl.ds(0, 16)])
    npair = (count + (2 * G - 1)) // (2 * G)

    def accum(ch, rbuf):
        def abody(q, carry):
            for u in range(UNR):
                r = q * UNR + u
                dlx = plsc.load_gather(
                    dlist, [jnp.full((16,), ch * G + r, jnp.int32)])
                pbase = dlx * 16 + iota
                for v in range(D // 16):
                    vals = rbuf[r, pl.ds(v * 16, 16)]
                    old = plsc.load_gather(accs[v], [pbase])
                    plsc.store_scatter(accs[v], [pbase], old + vals)
            return carry
        lax.fori_loop(0, G // UNR, abody, 0)

    def start(ch, rbuf, sem):
        pltpu.async_copy(hw_ref.at[slist.at[pl.ds(ch * G, G)]], rbuf, sem)

    start(0, rows0, sem0)

    def gpair(i, carry):
        start(2 * i + 1, rows1, sem1)
        pltpu.make_async_copy(hw_ref.at[pl.ds(0, G)], rows0, sem0).wait()
        accum(2 * i, rows0)

        @pl.when(2 * i + 2 < 2 * npair)
        def _():
            start(2 * i + 2, rows0, sem0)

        pltpu.make_async_copy(hw_ref.at[pl.ds(0, G)], rows1, sem1).wait()
        accum(2 * i + 1, rows1)
        return carry

    lax.fori_loop(0, npair, gpair, 0)

    for k in range(BLK // 40):
        def dbody(rr, carry):
            r = k * 40 + rr
            for v in range(D // 16):
                rows0[rr, pl.ds(v * 16, 16)] = accs[v][pl.ds(r * 16, 16)]
            return carry

        lax.fori_loop(0, 40, dbody, 0)
        pltpu.sync_copy(rows0.at[pl.ds(0, 40)],
                        out_ref.at[pl.ds(rowbase + k * 40, 40)])


_sc_call = pl.kernel(
    _sc_body,
    out_type=jax.ShapeDtypeStruct((NPAD, D), jnp.float32),
    mesh=_mesh,
    compiler_params=_sc_params,
    scratch_types=[
        pltpu.VMEM((CAP2,), jnp.int32),
        pltpu.VMEM((CAP2,), jnp.int32),
        pltpu.VMEM((16,), jnp.int32),
        pltpu.VMEM((G, D), jnp.float32),
        pltpu.VMEM((G, D), jnp.float32),
        pltpu.SemaphoreType.DMA,
        pltpu.SemaphoreType.DMA,
    ] + [pltpu.VMEM((BLK * 16,), jnp.float32)] * (D // 16),
)


def _k4_body(x_ref, a_ref, hw_ref, dh_ref, b_ref, o_ref):
    deg = jnp.sum(dh_ref[...], axis=1, keepdims=True) + 1.0
    dinv = lax.rsqrt(deg)
    o_ref[...] = (x_ref[...] + dinv * (a_ref[...] + hw_ref[...])
                  + b_ref[...])


def _k4_call(xpad, agg, hw, degh, b2):
    grid = (NPAD // _BN,)
    return pl.pallas_call(
        _k4_body,
        grid=grid,
        in_specs=[
            pl.BlockSpec((_BN, D), lambda i: (i, 0)),
            pl.BlockSpec((_BN, D), lambda i: (i, 0)),
            pl.BlockSpec((_BN, D), lambda i: (i, 0)),
            pl.BlockSpec((_BN, 16), lambda i: (i, 0)),
            pl.BlockSpec((1, D), lambda i: (0, 0)),
        ],
        out_specs=pl.BlockSpec((_BN, D), lambda i: (i, 0)),
        out_shape=jax.ShapeDtypeStruct((NPAD, D), jnp.float32),
    )(xpad, agg, hw, degh, b2)


@jax.jit
def kernel(x, edge_index, W, b, gamma, beta):
    src = edge_index[0]
    dst = edge_index[1]
    xpad = jnp.pad(x, ((0, NPAD - N), (0, 0)))
    src4 = src.reshape(ESEC, EROW, EK)
    dst4 = dst.reshape(ESEC, EROW, EK)

    degh, sl, dl, cnt = _scan_call(src4, dst4)
    degh = degh.reshape(NPAD, 16)
    hw = _k2_call(xpad, degh, gamma[None], beta[None], W)
    agg = _sc_call(hw, sl, dl, cnt)
    out = _k4_call(xpad, agg, hw, degh, b[None])
    return out[:N]
